# baseline (speedup 1.0000x reference)
import jax
import jax.numpy as jnp
from jax import lax
from jax.experimental import pallas as pl
from jax.experimental.pallas import tpu as pltpu, tpu_sc as plsc

NUM_CLASSES = 150
H = 512
W = 512
NPIX = H * W
NC = 2
NS = 16
NWORKERS = NC * NS
PIX_PER_WORKER = NPIX // NWORKERS
L = 16

CB = 25
NB = NUM_CLASSES // CB
CHUNK = 2048
NCHUNKS = PIX_PER_WORKER // CHUNK
NTILES = NCHUNKS * NB


def _one_hot_body(x_hbm, out_hbm):
    del x_hbm, out_hbm


@jax.jit
def _one_hot(x):
    k = pl.kernel(
        _one_hot_body,
        out_type=jax.ShapeDtypeStruct((NUM_CLASSES * NPIX,), jnp.int32),
        mesh=plsc.VectorSubcoreMesh(core_axis_name="c", subcore_axis_name="s"),
        compiler_params=pltpu.CompilerParams(
            use_tc_tiling_on_sc=False, needs_layout_passes=False,
            skip_device_barrier=True, disable_bounds_checks=True,
            disable_semaphore_checks=True),
    )
    return k(x)


def kernel(img):
    x = img.astype(jnp.int32).reshape(NPIX)
    out = _one_hot(x)
    return out.reshape(NUM_CLASSES, H, W)


# speedup vs baseline: 2.1170x; 2.1170x over previous
import jax
import jax.numpy as jnp
from jax import lax
from jax.experimental import pallas as pl
from jax.experimental.pallas import tpu as pltpu, tpu_sc as plsc

NUM_CLASSES = 150
H = 512
W = 512
NPIX = H * W
NC = 2
NS = 16
NWORKERS = NC * NS
L = 16

HG = H // 8
WG = W // 128
K = 4
CB = 38
HGB = HG // (NWORKERS // K)
NTILES = HGB * WG
TILE_PIX = 8 * 128


def _one_hot_body(x_hbm, out_hbm):
    cid = lax.axis_index("c")
    sid = lax.axis_index("s")
    wid = sid * NC + cid
    hg_blk = wid // K
    kblk = wid % K
    clo = jnp.where(kblk == K - 1, NUM_CLASSES - CB, kblk * CB)

    def inner(lab0, lab1, buf0, buf1, sem0, sem1):
        labs = (lab0, lab1)
        bufs = (buf0, buf1)
        sems = (sem0, sem1)

        zrow = jnp.zeros((L,), jnp.int32)

        def zero_body(c, carry):
            for buf in bufs:
                for j in range(TILE_PIX // L):
                    buf[c, pl.ds(j * L, L)] = zrow
            return carry
        lax.fori_loop(0, CB, zero_body, 0)

        lane = lax.iota(jnp.int32, L)

        def scatter_tile(buf, lab_v, wg, value):
            val = jnp.full((L,), value, jnp.int32)

            def body(j, carry):
                for u in range(4):
                    v = j * 4 + u
                    lab = lab_v[pl.ds(wg * TILE_PIX + v * L, L)]
                    m = (lab >= clo) & (lab < clo + CB)
                    row = jnp.where(m, lab - clo, 0)
                    plsc.store_scatter(
                        buf, [row, v * L + lane], val, mask=m)
                return carry
            lax.fori_loop(0, (TILE_PIX // L) // 4, body, 0)

        def tile_dst(t):
            hg = hg_blk * HGB + (t // WG)
            return out_hbm.at[pl.ds(clo, CB), hg,
                              pl.ds((t % WG) * TILE_PIX, TILE_PIX)]

        copies = [None, None]
        params = [None, None]
        for t in range(NTILES):
            b = t % 2
            lp = (t // WG) % 2
            if t % WG == 0:
                hg = hg_blk * HGB + (t // WG)
                pltpu.sync_copy(x_hbm.at[hg], labs[lp])
            if t >= 2:
                copies[b].wait()
                plab, pwg = params[b]
                scatter_tile(bufs[b], labs[plab], pwg, 0)
            scatter_tile(bufs[b], labs[lp], t % WG, 1)
            copies[b] = pltpu.async_copy(bufs[b], tile_dst(t), sems[b])
            params[b] = (lp, t % WG)
        copies[0].wait()
        copies[1].wait()

    pl.run_scoped(
        inner,
        pltpu.VMEM((WG * TILE_PIX,), jnp.int32),
        pltpu.VMEM((WG * TILE_PIX,), jnp.int32),
        pltpu.VMEM((CB, TILE_PIX), jnp.int32),
        pltpu.VMEM((CB, TILE_PIX), jnp.int32),
        pltpu.SemaphoreType.DMA,
        pltpu.SemaphoreType.DMA,
    )


@jax.jit
def _one_hot(x):
    k = pl.kernel(
        _one_hot_body,
        out_type=jax.ShapeDtypeStruct((NUM_CLASSES, HG, WG * TILE_PIX),
                                      jnp.int32),
        mesh=plsc.VectorSubcoreMesh(core_axis_name="c", subcore_axis_name="s"),
        compiler_params=pltpu.CompilerParams(
            use_tc_tiling_on_sc=False, needs_layout_passes=False),
    )
    return k(x)


def kernel(img):
    x = (img.astype(jnp.int32)
         .reshape(HG, 8, WG, 128)
         .transpose(0, 2, 1, 3)
         .reshape(HG, WG * TILE_PIX))
    out = _one_hot(x)
    return (out.reshape(NUM_CLASSES, HG, WG, 8, 128)
            .transpose(0, 1, 3, 2, 4)
            .reshape(NUM_CLASSES, H, W))
